# initial kernel scaffold (unmeasured)
import jax
import jax.numpy as jnp
from jax import lax
from jax.experimental import pallas as pl
from jax.experimental.pallas import tpu as pltpu

B, H, D, BS = 32, 16, 128, 32
NB_LOCAL = 256
NB_SLOTS = 256
SCALE = D ** -0.5
NEG = -1e30


def kernel(Q, K, V, bt, lens):
    def body(Q_ref, K_hbm, V_hbm, bt_ref, lens_ref, out_ref,
             o_acc, stats, o_rcv, stats_rcv, kbuf, vbuf,
             copy_sems, send_sems, recv_sems):
        my_x = lax.axis_index("x")
        my_y = lax.axis_index("y")
        my_z = lax.axis_index("z")
        lo = my_y * NB_LOCAL

        stats[0, :, :] = jnp.full((B, H), NEG, jnp.float32)
        stats[1, :, :] = jnp.zeros((B, H), jnp.float32)
        o_acc[...] = jnp.zeros((B, H, D), jnp.float32)

        def page_step(b, j):
            page = bt_ref[b, j]
            local = page - lo
            valid = (j < lens_ref[b]) & (local >= 0) & (local < NB_LOCAL)

            @pl.when(valid)
            def _():
                ck = pltpu.make_async_copy(K_hbm.at[local], kbuf, copy_sems.at[0])
                cv = pltpu.make_async_copy(V_hbm.at[local], vbuf, copy_sems.at[1])
                ck.start()
                cv.start()
                ck.wait()
                cv.wait()
                q = Q_ref[b, 0]
                s = jnp.sum(q[None, :, :] * kbuf[...], axis=-1) * SCALE
                m_old = stats[0, pl.ds(b, 1), :]
                m_new = jnp.maximum(m_old, jnp.max(s, axis=0, keepdims=True))
                corr = jnp.exp(m_old - m_new)
                p = jnp.exp(s - m_new)
                stats[0, pl.ds(b, 1), :] = m_new
                stats[1, pl.ds(b, 1), :] = (
                    stats[1, pl.ds(b, 1), :] * corr
                    + jnp.sum(p, axis=0, keepdims=True)
                )
                pv = jnp.sum(p[:, :, None] * vbuf[...], axis=0)
                o_acc[b] = o_acc[b] * corr.reshape(H, 1) + pv

        def batch_loop(b, carry):
            def page_loop(j, c2):
                page_step(b, j)
                return c2
            lax.fori_loop(0, NB_SLOTS, page_loop, 0)
            return carry

        lax.fori_loop(0, B, batch_loop, 0)

        nbr = (my_x, 1 - my_y, my_z)
        bar = pltpu.get_barrier_semaphore()
        pl.semaphore_signal(bar, inc=1, device_id=nbr,
                            device_id_type=pl.DeviceIdType.MESH)
        pl.semaphore_wait(bar, 1)

        r_o = pltpu.make_async_remote_copy(
            src_ref=o_acc, dst_ref=o_rcv,
            send_sem=send_sems.at[0], recv_sem=recv_sems.at[0],
            device_id=nbr, device_id_type=pl.DeviceIdType.MESH)
        r_s = pltpu.make_async_remote_copy(
            src_ref=stats, dst_ref=stats_rcv,
            send_sem=send_sems.at[1], recv_sem=recv_sems.at[1],
            device_id=nbr, device_id_type=pl.DeviceIdType.MESH)
        r_o.start()
        r_s.start()
        r_o.wait()
        r_s.wait()

        m_a = stats[0, :, :]
        l_a = stats[1, :, :]
        m_b = stats_rcv[0, :, :]
        l_b = stats_rcv[1, :, :]
        m = jnp.maximum(m_a, m_b)
        ca = jnp.exp(m_a - m)
        cb = jnp.exp(m_b - m)
        l = l_a * ca + l_b * cb
        o = o_acc[...] * ca[:, :, None] + o_rcv[...] * cb[:, :, None]
        out = o / l[:, :, None]
        out_ref[...] = out[:, None, :, :]

    return pl.pallas_call(
        body,
        out_shape=jax.ShapeDtypeStruct((B, 1, H, D), jnp.float32),
        in_specs=[
            pl.BlockSpec(memory_space=pltpu.VMEM),
            pl.BlockSpec(memory_space=pltpu.ANY),
            pl.BlockSpec(memory_space=pltpu.ANY),
            pl.BlockSpec(memory_space=pltpu.SMEM),
            pl.BlockSpec(memory_space=pltpu.SMEM),
        ],
        out_specs=pl.BlockSpec(memory_space=pltpu.VMEM),
        scratch_shapes=[
            pltpu.VMEM((B, H, D), jnp.float32),
            pltpu.VMEM((2, B, H), jnp.float32),
            pltpu.VMEM((B, H, D), jnp.float32),
            pltpu.VMEM((2, B, H), jnp.float32),
            pltpu.VMEM((BS, H, D), jnp.float32),
            pltpu.VMEM((BS, H, D), jnp.float32),
            pltpu.SemaphoreType.DMA((2,)),
            pltpu.SemaphoreType.DMA((2,)),
            pltpu.SemaphoreType.DMA((2,)),
        ],
        compiler_params=pltpu.CompilerParams(collective_id=0),
    )(Q, K, V, bt, lens)


# baseline (device time: 3303964 ns/iter reference)
import jax
import jax.numpy as jnp
from jax import lax
from jax.experimental import pallas as pl
from jax.experimental.pallas import tpu as pltpu

B, H, D, BS = 32, 16, 128, 32
NB_LOCAL = 256
NB_SLOTS = 256
SCALE = D ** -0.5
NEG = -1e30


def kernel(Q, K, V, bt, lens):
    def body(Q_ref, K_hbm, V_hbm, bt_ref, lens_ref, out_ref,
             o_acc, stats, o_rcv, stats_rcv, kbuf, vbuf,
             copy_sems, send_sems, recv_sems):
        my_x = lax.axis_index("x")
        my_y = lax.axis_index("y")
        my_z = lax.axis_index("z")
        lo = my_y * NB_LOCAL

        stats[0, :, :] = jnp.full((B, H), NEG, jnp.float32)
        stats[1, :, :] = jnp.zeros((B, H), jnp.float32)
        o_acc[...] = jnp.zeros((B, H, D), jnp.float32)

        def page_step(b, j):
            page = bt_ref[b, j]
            local = page - lo
            valid = (j < lens_ref[b]) & (local >= 0) & (local < NB_LOCAL)

            @pl.when(valid)
            def _():
                ck = pltpu.make_async_copy(K_hbm.at[local], kbuf, copy_sems.at[0])
                cv = pltpu.make_async_copy(V_hbm.at[local], vbuf, copy_sems.at[1])
                ck.start()
                cv.start()
                ck.wait()
                cv.wait()
                q = Q_ref[b, 0]
                s = jnp.sum(q[None, :, :] * kbuf[...], axis=-1) * SCALE
                m_old = stats[0, pl.ds(b, 1), :]
                m_new = jnp.maximum(m_old, jnp.max(s, axis=0, keepdims=True))
                corr = jnp.exp(m_old - m_new)
                p = jnp.exp(s - m_new)
                stats[0, pl.ds(b, 1), :] = m_new
                stats[1, pl.ds(b, 1), :] = (
                    stats[1, pl.ds(b, 1), :] * corr
                    + jnp.sum(p, axis=0, keepdims=True)
                )
                pv = jnp.sum(p[:, :, None] * vbuf[...], axis=0)
                o_acc[b] = o_acc[b] * corr.reshape(H, 1) + pv

        def batch_loop(b, carry):
            def page_loop(j, c2):
                page_step(b, j)
                return c2
            lax.fori_loop(0, NB_SLOTS, page_loop, 0)
            return carry

        lax.fori_loop(0, B, batch_loop, 0)

        nbr = (my_x, 1 - my_y, my_z)
        bar = pltpu.get_barrier_semaphore()
        pl.semaphore_signal(bar, inc=1, device_id=nbr,
                            device_id_type=pl.DeviceIdType.MESH)
        pl.semaphore_wait(bar, 1)

        r_o = pltpu.make_async_remote_copy(
            src_ref=o_acc, dst_ref=o_rcv,
            send_sem=send_sems.at[0], recv_sem=recv_sems.at[0],
            device_id=nbr, device_id_type=pl.DeviceIdType.MESH)
        r_s = pltpu.make_async_remote_copy(
            src_ref=stats, dst_ref=stats_rcv,
            send_sem=send_sems.at[1], recv_sem=recv_sems.at[1],
            device_id=nbr, device_id_type=pl.DeviceIdType.MESH)
        r_o.start()
        r_s.start()
        r_o.wait()
        r_s.wait()

        m_a = stats[0, :, :]
        l_a = stats[1, :, :]
        m_b = stats_rcv[0, :, :]
        l_b = stats_rcv[1, :, :]
        m = jnp.maximum(m_a, m_b)
        ca = jnp.exp(m_a - m)
        cb = jnp.exp(m_b - m)
        l = l_a * ca + l_b * cb
        o = o_acc[...] * ca[:, :, None] + o_rcv[...] * cb[:, :, None]
        out = o / l[:, :, None]
        out_ref[...] = out[:, None, :, :]

    return pl.pallas_call(
        body,
        out_shape=jax.ShapeDtypeStruct((B, 1, H, D), jnp.float32),
        in_specs=[
            pl.BlockSpec(memory_space=pltpu.VMEM),
            pl.BlockSpec(memory_space=pltpu.MemorySpace.HBM),
            pl.BlockSpec(memory_space=pltpu.MemorySpace.HBM),
            pl.BlockSpec(memory_space=pltpu.MemorySpace.SMEM),
            pl.BlockSpec(memory_space=pltpu.MemorySpace.SMEM),
        ],
        out_specs=pl.BlockSpec(memory_space=pltpu.VMEM),
        scratch_shapes=[
            pltpu.VMEM((B, H, D), jnp.float32),
            pltpu.VMEM((2, B, H), jnp.float32),
            pltpu.VMEM((B, H, D), jnp.float32),
            pltpu.VMEM((2, B, H), jnp.float32),
            pltpu.VMEM((BS, H, D), jnp.float32),
            pltpu.VMEM((BS, H, D), jnp.float32),
            pltpu.SemaphoreType.DMA((2,)),
            pltpu.SemaphoreType.DMA((2,)),
            pltpu.SemaphoreType.DMA((2,)),
        ],
        compiler_params=pltpu.CompilerParams(collective_id=0),
    )(Q, K, V, bt, lens)


# device time: 313201 ns/iter; 10.5490x vs baseline; 10.5490x over previous
import jax
import jax.numpy as jnp
from jax import lax
from jax.experimental import pallas as pl
from jax.experimental.pallas import tpu as pltpu

B, H, D, BS = 32, 16, 128, 32
NB_LOCAL = 256
NB_SLOTS = 256
BPD = B // 4
NSLOT = 4
SCALE = D ** -0.5
NEG = -1e30


def kernel(Q, K, V, bt, lens):
    my_y_out = lax.axis_index("y")
    lo = (my_y_out * NB_LOCAL).astype(jnp.int32)
    j = jnp.arange(NB_SLOTS, dtype=jnp.int32)[None, :]
    local = bt - lo
    valid = (j < lens[:, None]) & (local >= 0) & (local < NB_LOCAL)
    order = jnp.argsort(jnp.where(valid, j, NB_SLOTS), axis=1)
    comp = jnp.take_along_axis(local, order, axis=1).astype(jnp.int32)
    counts = valid.sum(axis=1).astype(jnp.int32)

    def body(Q_ref, K_hbm, V_hbm, comp_ref, counts_ref, out_ref,
             o_part, stats, o_rcv, stats_rcv, ag_buf, kbuf, vbuf,
             kc_sems, vc_sems, send_sems, recv_sems):
        my_x = lax.axis_index("x")
        my_y = lax.axis_index("y")
        my_z = lax.axis_index("z")
        qid = my_x * 2 + my_z
        b0 = qid * BPD

        y_nbr = (my_x, 1 - my_y, my_z)
        z_nbr = (my_x, my_y, 1 - my_z)
        x_nbr = (1 - my_x, my_y, my_z)

        bar = pltpu.get_barrier_semaphore()
        for nbr in (y_nbr, z_nbr, x_nbr):
            pl.semaphore_signal(bar, inc=1, device_id=nbr,
                                device_id_type=pl.DeviceIdType.MESH)
        pl.semaphore_wait(bar, 3)

        def start_copy(slot, page):
            pltpu.make_async_copy(K_hbm.at[page], kbuf.at[slot],
                                  kc_sems.at[slot]).start()
            pltpu.make_async_copy(V_hbm.at[page], vbuf.at[slot],
                                  vc_sems.at[slot]).start()

        def do_batch(bb, carry):
            b = b0 + bb
            nt = counts_ref[b]
            q = Q_ref[b, 0]

            for t0 in range(NSLOT - 1):
                @pl.when(nt > t0)
                def _():
                    start_copy(t0, comp_ref[b, t0])

            def step(t, mlo):
                m, l, o = mlo
                slot = lax.rem(t, NSLOT)
                pltpu.make_async_copy(K_hbm.at[0], kbuf.at[slot],
                                      kc_sems.at[slot]).wait()
                pltpu.make_async_copy(V_hbm.at[0], vbuf.at[slot],
                                      vc_sems.at[slot]).wait()

                @pl.when(t + NSLOT - 1 < nt)
                def _():
                    start_copy(lax.rem(t + NSLOT - 1, NSLOT),
                               comp_ref[b, t + NSLOT - 1])

                kp = kbuf[slot]
                vp = vbuf[slot]
                s = jnp.sum(q[None, :, :] * kp, axis=-1) * SCALE
                m_new = jnp.maximum(m, jnp.max(s, axis=0, keepdims=True))
                corr = jnp.exp(m - m_new)
                p = jnp.exp(s - m_new)
                l_new = l * corr + jnp.sum(p, axis=0, keepdims=True)
                pv = jnp.sum(p[:, :, None] * vp, axis=0)
                o_new = o * corr.reshape(H, 1) + pv
                return (m_new, l_new, o_new)

            init = (jnp.full((1, H), NEG, jnp.float32),
                    jnp.zeros((1, H), jnp.float32),
                    jnp.zeros((H, D), jnp.float32))
            m, l, o = lax.fori_loop(0, nt, step, init)
            stats[0, pl.ds(bb, 1), :] = m
            stats[1, pl.ds(bb, 1), :] = l
            o_part[bb] = o
            return carry

        lax.fori_loop(0, BPD, do_batch, 0)

        r_o = pltpu.make_async_remote_copy(
            src_ref=o_part, dst_ref=o_rcv,
            send_sem=send_sems.at[0], recv_sem=recv_sems.at[0],
            device_id=y_nbr, device_id_type=pl.DeviceIdType.MESH)
        r_s = pltpu.make_async_remote_copy(
            src_ref=stats, dst_ref=stats_rcv,
            send_sem=send_sems.at[1], recv_sem=recv_sems.at[1],
            device_id=y_nbr, device_id_type=pl.DeviceIdType.MESH)
        r_o.start()
        r_s.start()
        r_o.wait()
        r_s.wait()

        m_a = stats[0, :, :]
        l_a = stats[1, :, :]
        m_b = stats_rcv[0, :, :]
        l_b = stats_rcv[1, :, :]
        m = jnp.maximum(m_a, m_b)
        ca = jnp.exp(m_a - m)
        cb = jnp.exp(m_b - m)
        l = l_a * ca + l_b * cb
        o = o_part[...] * ca[:, :, None] + o_rcv[...] * cb[:, :, None]
        ag_buf[pl.ds(b0, BPD)] = o / l[:, :, None]

        r_z = pltpu.make_async_remote_copy(
            src_ref=ag_buf.at[pl.ds(b0, BPD)],
            dst_ref=ag_buf.at[pl.ds(b0, BPD)],
            send_sem=send_sems.at[2], recv_sem=recv_sems.at[2],
            device_id=z_nbr, device_id_type=pl.DeviceIdType.MESH)
        r_z.start()
        r_z.wait()

        x0 = my_x * (2 * BPD)
        r_x = pltpu.make_async_remote_copy(
            src_ref=ag_buf.at[pl.ds(x0, 2 * BPD)],
            dst_ref=ag_buf.at[pl.ds(x0, 2 * BPD)],
            send_sem=send_sems.at[3], recv_sem=recv_sems.at[3],
            device_id=x_nbr, device_id_type=pl.DeviceIdType.MESH)
        r_x.start()
        r_x.wait()

        out_ref[...] = ag_buf[...][:, None, :, :]

    return pl.pallas_call(
        body,
        out_shape=jax.ShapeDtypeStruct((B, 1, H, D), jnp.float32),
        in_specs=[
            pl.BlockSpec(memory_space=pltpu.MemorySpace.VMEM),
            pl.BlockSpec(memory_space=pltpu.MemorySpace.HBM),
            pl.BlockSpec(memory_space=pltpu.MemorySpace.HBM),
            pl.BlockSpec(memory_space=pltpu.MemorySpace.SMEM),
            pl.BlockSpec(memory_space=pltpu.MemorySpace.SMEM),
        ],
        out_specs=pl.BlockSpec(memory_space=pltpu.MemorySpace.VMEM),
        scratch_shapes=[
            pltpu.VMEM((BPD, H, D), jnp.float32),
            pltpu.VMEM((2, BPD, H), jnp.float32),
            pltpu.VMEM((BPD, H, D), jnp.float32),
            pltpu.VMEM((2, BPD, H), jnp.float32),
            pltpu.VMEM((B, H, D), jnp.float32),
            pltpu.VMEM((NSLOT, BS, H, D), jnp.float32),
            pltpu.VMEM((NSLOT, BS, H, D), jnp.float32),
            pltpu.SemaphoreType.DMA((NSLOT,)),
            pltpu.SemaphoreType.DMA((NSLOT,)),
            pltpu.SemaphoreType.DMA((4,)),
            pltpu.SemaphoreType.DMA((4,)),
        ],
        compiler_params=pltpu.CompilerParams(collective_id=0),
    )(Q, K, V, comp, counts)
